# baseline (speedup 1.0000x reference)
import jax
import jax.numpy as jnp
from jax import lax
from jax.experimental import pallas as pl
from jax.experimental.pallas import tpu as pltpu
from jax.experimental.pallas import tpu_sc as plsc

B = 16384
NCAND = 20
D = 64
LANES = 16
NC = 2
NS = 16
NW = NC * NS
BPW = B // NW
CPW = BPW * NCAND
CHUNK = 128
NCHUNK = CPW // CHUNK
UCHUNK = BPW // CHUNK
GROUPS = CHUNK // LANES


def _dot_body(user_idx, item_idx, utab, itab, out,
              uidx_v, iidx_v, urows_v, irows_v, scores_v, sem):
    cid = lax.axis_index("c")
    sid = lax.axis_index("s")
    wid = sid * NC + cid

    pltpu.sync_copy(user_idx.at[pl.ds(wid * UCHUNK, UCHUNK)], uidx_v)
    pltpu.sync_copy(item_idx.at[pl.ds(wid * NCHUNK, NCHUNK)], iidx_v)

    for j in range(UCHUNK):
        pltpu.async_copy(utab.at[uidx_v.at[j]],
                         urows_v.at[pl.ds(j * CHUNK, CHUNK)], sem).wait()

    lane = lax.iota(jnp.int32, LANES)

    def chunk_body(j, carry):
        pltpu.async_copy(itab.at[iidx_v.at[j]], irows_v, sem).wait()
        c_base = j * CHUNK
        for g in range(GROUPS):
            acc = jnp.zeros((LANES,), jnp.float32)
            for k in range(LANES):
                r = g * LANES + k
                b = (c_base + r) // NCAND
                t = (urows_v[b, pl.ds(0, 16)] * irows_v[r, pl.ds(0, 16)]
                     + urows_v[b, pl.ds(16, 16)] * irows_v[r, pl.ds(16, 16)]
                     + urows_v[b, pl.ds(32, 16)] * irows_v[r, pl.ds(32, 16)]
                     + urows_v[b, pl.ds(48, 16)] * irows_v[r, pl.ds(48, 16)])
                acc = jnp.where(lane == k, jnp.sum(t), acc)
            scores_v[pl.ds(c_base + g * LANES, LANES)] = acc
        return carry

    lax.fori_loop(0, NCHUNK, chunk_body, 0)
    pltpu.sync_copy(scores_v, out.at[pl.ds(wid * CPW, CPW)])


def kernel(user, item, user_table, item_table):
    user2d = user.reshape(B // CHUNK, CHUNK)
    item2d = item.reshape((B * NCAND) // CHUNK, CHUNK)
    mesh = plsc.VectorSubcoreMesh(core_axis_name="c", subcore_axis_name="s")
    scores = pl.kernel(
        _dot_body,
        out_type=jax.ShapeDtypeStruct((B * NCAND,), jnp.float32),
        mesh=mesh,
        compiler_params=pltpu.CompilerParams(
            needs_layout_passes=False, use_tc_tiling_on_sc=False),
        scratch_types=[
            pltpu.VMEM((UCHUNK, CHUNK), jnp.int32),
            pltpu.VMEM((NCHUNK, CHUNK), jnp.int32),
            pltpu.VMEM((BPW, D), jnp.float32),
            pltpu.VMEM((CHUNK, D), jnp.float32),
            pltpu.VMEM((CPW,), jnp.float32),
            pltpu.SemaphoreType.DMA,
        ],
    )(user2d, item2d, user_table, item_table)
    return scores.reshape(B, NCAND)
